# baseline (speedup 1.0000x reference)
import math

import jax
import jax.numpy as jnp
from jax import lax
from jax.experimental import pallas as pl
from jax.experimental.pallas import tpu as pltpu

B = 16
L = 2048
NB = 6
EMB = 64
NH = 16
TMAX = 1500.0

N = NB * L
NPAD = 16384
NROW = NPAD // 128
BIG = 9999999.0
PADKEY = 1.0e10
LOG2N = 14

TOK = 1024
NT = N // TOK


def _sort_body(t_ref, m_ref, x_ref, ts_ref, ms_ref, xs_ref, idx_ref):
    tv = t_ref[0]
    mv = m_ref[0]
    xv = x_ref[0]
    r = lax.broadcasted_iota(jnp.int32, (NROW, 128), 0)
    c = lax.broadcasted_iota(jnp.int32, (NROW, 128), 1)
    j = r * 128 + c

    key = jnp.where(mv > 0.0, tv, BIG)
    key = jnp.where(j < N, key, PADKEY)
    idx = j
    xc = xv
    tc = tv

    for k in range(1, LOG2N + 1):
        ksize = 1 << k
        for s in range(k - 1, -1, -1):
            d = 1 << s
            if d < 128:
                bit = (c & d) != 0
                axis, sh = 1, d
            else:
                dr = d // 128
                bit = (r & dr) != 0
                axis, sh = 0, dr

            def xorshuf(A, bit=bit, axis=axis, sh=sh):
                return jnp.where(bit, jnp.roll(A, sh, axis=axis),
                                 jnp.roll(A, -sh, axis=axis))

            pk = xorshuf(key)
            pi = xorshuf(idx)
            px = xorshuf(xc)
            pt = xorshuf(tc)
            a_lt_b = (key < pk) | ((key == pk) & (idx < pi))
            up = (j & ksize) == 0
            i_low = (j & d) == 0
            want_min = up == i_low
            take = a_lt_b ^ want_min
            key = jnp.where(take, pk, key)
            idx = jnp.where(take, pi, idx)
            xc = jnp.where(take, px, xc)
            tc = jnp.where(take, pt, tc)

    ts_ref[0] = tc
    xs_ref[0] = xc
    idx_ref[0] = idx
    ms_ref[0] = jnp.where(key < 5.0e6, 1.0, 0.0)


def _film_body(t_ref, x_ref, i_ref, W_ref, lw_ref, o_ref):
    tcol = t_ref[0]
    xcol = x_ref[0]
    band = i_ref[0] >> 11
    lane = lax.broadcasted_iota(jnp.int32, (TOK, 2 * NH), 1)
    hv = ((lane % NH) + 1).astype(jnp.float32) * (2.0 * math.pi / TMAX)
    th = tcol * hv
    F = jnp.where(lane < NH, jnp.sin(th), jnp.cos(th))
    acc = jnp.zeros((TOK, 2 * EMB), jnp.float32)
    lsel = jnp.zeros((TOK, EMB), jnp.float32)
    for i in range(NB):
        mi = (band == i).astype(jnp.float32)
        acc = acc + jnp.dot(F * mi, W_ref[i],
                            preferred_element_type=jnp.float32,
                            precision=lax.Precision.HIGHEST)
        lsel = lsel + mi * lw_ref[i]
    o_ref[0] = xcol * lsel * acc[:, :EMB] + acc[:, EMB:]


def kernel(x, t, mask, lin_w, a, b, w, v):
    tt = t.transpose(0, 2, 1).reshape(B, N)
    xx = x.transpose(0, 2, 1).reshape(B, N)
    mm = mask.transpose(0, 2, 1).reshape(B, N)
    pad = ((0, 0), (0, NPAD - N))
    tt = jnp.pad(tt, pad).reshape(B, NROW, 128)
    xx = jnp.pad(xx, pad).reshape(B, NROW, 128)
    mm = jnp.pad(mm, pad).reshape(B, NROW, 128)

    blk = pl.BlockSpec((1, NROW, 128), lambda bb: (bb, 0, 0))
    f32 = jnp.float32
    ts, ms, xs, idxs = pl.pallas_call(
        _sort_body,
        grid=(B,),
        in_specs=[blk, blk, blk],
        out_specs=[blk, blk, blk, blk],
        out_shape=[
            jax.ShapeDtypeStruct((B, NROW, 128), f32),
            jax.ShapeDtypeStruct((B, NROW, 128), f32),
            jax.ShapeDtypeStruct((B, NROW, 128), f32),
            jax.ShapeDtypeStruct((B, NROW, 128), jnp.int32),
        ],
    )(tt, mm, xx)

    ts2 = ts.reshape(B, NPAD)[:, :N]
    ms2 = ms.reshape(B, NPAD)[:, :N]
    xs2 = xs.reshape(B, NPAD)[:, :N]
    is2 = idxs.reshape(B, NPAD)[:, :N]

    Wg = jnp.concatenate([a, w], axis=1)
    Wb = jnp.concatenate([b, v], axis=1)
    Wcat = jnp.concatenate([Wg, Wb], axis=2)
    lwv = lin_w[:, :, 0]

    tok_blk = pl.BlockSpec((1, TOK, 1), lambda bb, tb: (bb, tb, 0))
    x_out = pl.pallas_call(
        _film_body,
        grid=(B, NT),
        in_specs=[
            tok_blk, tok_blk, tok_blk,
            pl.BlockSpec((NB, 2 * NH, 2 * EMB), lambda bb, tb: (0, 0, 0)),
            pl.BlockSpec((NB, EMB), lambda bb, tb: (0, 0)),
        ],
        out_specs=pl.BlockSpec((1, TOK, EMB), lambda bb, tb: (bb, tb, 0)),
        out_shape=jax.ShapeDtypeStruct((B, N, EMB), f32),
    )(ts2[..., None], xs2[..., None], is2[..., None], Wcat, lwv)

    return (x_out, ms2[..., None], ts2[..., None])


# speedup vs baseline: 2.6981x; 2.6981x over previous
import math

import jax
import jax.numpy as jnp
from jax import lax
from jax.experimental import pallas as pl
from jax.experimental.pallas import tpu as pltpu

B = 16
L = 2048
NB = 6
EMB = 64
NH = 16
TMAX = 1500.0

N = NB * L
NPAD = 16384
NROW = NPAD // 128
BIG = 9999999.0
PADKEY = 1.0e10
LOG2N = 14

TOK = 1024
NT = N // TOK


def _sort_body(t_ref, m_ref, x_ref, ts_ref, ms_ref, xs_ref, idx_ref):
    tv = t_ref[0]
    mv = m_ref[0]
    xv = x_ref[0]
    r = lax.broadcasted_iota(jnp.int32, (NROW, 128), 0)
    c = lax.broadcasted_iota(jnp.int32, (NROW, 128), 1)
    j = r * 128 + c

    key = jnp.where(mv > 0.0, tv, BIG)
    key = jnp.where(j < N, key, PADKEY)
    idx = j
    xc = xv
    tc = tv

    for k in range(1, LOG2N + 1):
        ksize = 1 << k
        for s in range(k - 1, -1, -1):
            d = 1 << s
            if d < 128:
                bit = (c & d) != 0
                axis, sh = 1, d
            else:
                dr = d // 128
                bit = (r & dr) != 0
                axis, sh = 0, dr

            def xorshuf(A, bit=bit, axis=axis, sh=sh):
                return jnp.where(bit, jnp.roll(A, sh, axis=axis),
                                 jnp.roll(A, -sh, axis=axis))

            pk = xorshuf(key)
            pi = xorshuf(idx)
            px = xorshuf(xc)
            pt = xorshuf(tc)
            a_lt_b = (key < pk) | ((key == pk) & (idx < pi))
            up = (j & ksize) == 0
            i_low = (j & d) == 0
            want_min = up == i_low
            take = a_lt_b ^ want_min
            key = jnp.where(take, pk, key)
            idx = jnp.where(take, pi, idx)
            xc = jnp.where(take, px, xc)
            tc = jnp.where(take, pt, tc)

    ts_ref[0] = tc
    xs_ref[0] = xc
    idx_ref[0] = idx
    ms_ref[0] = jnp.where(key < 5.0e6, 1.0, 0.0)


def _film_body(t_ref, x_ref, i_ref, W_ref, lw_ref, o_ref):
    trow = t_ref[0, 0]
    xrow = x_ref[0, 0]
    band = i_ref[0, 0] >> 11
    sub = lax.broadcasted_iota(jnp.int32, (2 * NH, TOK), 0)
    hv = ((sub % NH) + 1).astype(jnp.float32) * (2.0 * math.pi / TMAX)
    th = trow * hv
    F = jnp.where(sub < NH, jnp.sin(th), jnp.cos(th))
    acc = jnp.zeros((2 * EMB, TOK), jnp.float32)
    for i in range(NB):
        mi = (band == i).astype(jnp.float32)
        acc = acc + jnp.dot(W_ref[i], F * mi,
                            preferred_element_type=jnp.float32)
    bsub = lax.broadcasted_iota(jnp.int32, (8, TOK), 0)
    oh = (band == bsub).astype(jnp.float32)
    lsel = jnp.dot(lw_ref[...], oh, preferred_element_type=jnp.float32)
    outT = xrow * lsel * acc[:EMB, :] + acc[EMB:, :]
    o_ref[0] = outT.T


def kernel(x, t, mask, lin_w, a, b, w, v):
    tt = t.transpose(0, 2, 1).reshape(B, N)
    xx = x.transpose(0, 2, 1).reshape(B, N)
    mm = mask.transpose(0, 2, 1).reshape(B, N)
    pad = ((0, 0), (0, NPAD - N))
    tt = jnp.pad(tt, pad).reshape(B, NROW, 128)
    xx = jnp.pad(xx, pad).reshape(B, NROW, 128)
    mm = jnp.pad(mm, pad).reshape(B, NROW, 128)

    blk = pl.BlockSpec((1, NROW, 128), lambda bb: (bb, 0, 0))
    f32 = jnp.float32
    ts, ms, xs, idxs = pl.pallas_call(
        _sort_body,
        grid=(B,),
        in_specs=[blk, blk, blk],
        out_specs=[blk, blk, blk, blk],
        out_shape=[
            jax.ShapeDtypeStruct((B, NROW, 128), f32),
            jax.ShapeDtypeStruct((B, NROW, 128), f32),
            jax.ShapeDtypeStruct((B, NROW, 128), f32),
            jax.ShapeDtypeStruct((B, NROW, 128), jnp.int32),
        ],
    )(tt, mm, xx)

    ts2 = ts.reshape(B, NPAD)[:, :N]
    ms2 = ms.reshape(B, NPAD)[:, :N]
    xs2 = xs.reshape(B, NPAD)[:, :N]
    is2 = idxs.reshape(B, NPAD)[:, :N]

    Wg = jnp.concatenate([a, w], axis=1)
    Wb = jnp.concatenate([b, v], axis=1)
    Wcat = jnp.concatenate([Wg, Wb], axis=2)
    WcatT = Wcat.transpose(0, 2, 1)
    lwT = jnp.zeros((EMB, 8), f32).at[:, :NB].set(lin_w[:, :, 0].T)

    tok_blk = pl.BlockSpec((1, 1, 1, TOK), lambda bb, tb: (bb, tb, 0, 0))
    x_out = pl.pallas_call(
        _film_body,
        grid=(B, NT),
        in_specs=[
            tok_blk, tok_blk, tok_blk,
            pl.BlockSpec((NB, 2 * EMB, 2 * NH), lambda bb, tb: (0, 0, 0)),
            pl.BlockSpec((EMB, 8), lambda bb, tb: (0, 0)),
        ],
        out_specs=pl.BlockSpec((1, TOK, EMB), lambda bb, tb: (bb, tb, 0)),
        out_shape=jax.ShapeDtypeStruct((B, N, EMB), f32),
    )(ts2.reshape(B, NT, 1, TOK), xs2.reshape(B, NT, 1, TOK),
      is2.reshape(B, NT, 1, TOK), WcatT, lwT)

    return (x_out, ms2[..., None], ts2[..., None])


# speedup vs baseline: 2.9621x; 1.0979x over previous
import math

import jax
import jax.numpy as jnp
from jax import lax
from jax.experimental import pallas as pl
from jax.experimental.pallas import tpu as pltpu

B = 16
L = 2048
NB = 6
EMB = 64
NH = 16
TMAX = 1500.0

N = NB * L
NPAD = 16384
NROW = NPAD // 128
BIG = 9999999.0
PADKEY = 1.0e10
LOG2N = 14

TOK = 1024
NT = N // TOK


def _sort_body(t_ref, m_ref, x_ref, ts_ref, ms_ref, xs_ref, idx_ref):
    tv = t_ref[0]
    mv = m_ref[0]
    xv = x_ref[0]
    r = lax.broadcasted_iota(jnp.int32, (NROW, 128), 0)
    c = lax.broadcasted_iota(jnp.int32, (NROW, 128), 1)
    j = r * 128 + c

    key = jnp.where(mv > 0.0, tv, BIG)
    key = jnp.where(j < N, key, PADKEY)
    idx = j
    xc = xv
    tc = tv

    for k in range(1, LOG2N + 1):
        ksize = 1 << k
        for s in range(k - 1, -1, -1):
            d = 1 << s
            if d < 128:
                perm = c ^ d

                def xorshuf(A, perm=perm):
                    return jnp.take_along_axis(A, perm, axis=1)
            else:
                dr = d // 128
                bit = (r & dr) != 0

                def xorshuf(A, bit=bit, sh=dr):
                    return jnp.where(bit, jnp.roll(A, sh, axis=0),
                                     jnp.roll(A, -sh, axis=0))

            pk = xorshuf(key)
            pi = xorshuf(idx)
            px = xorshuf(xc)
            pt = xorshuf(tc)
            a_lt_b = (key < pk) | ((key == pk) & (idx < pi))
            up = (j & ksize) == 0
            i_low = (j & d) == 0
            want_min = up == i_low
            take = a_lt_b ^ want_min
            key = jnp.where(take, pk, key)
            idx = jnp.where(take, pi, idx)
            xc = jnp.where(take, px, xc)
            tc = jnp.where(take, pt, tc)

    ts_ref[0] = tc
    xs_ref[0] = xc
    idx_ref[0] = idx
    ms_ref[0] = jnp.where(key < 5.0e6, 1.0, 0.0)


def _film_body(t_ref, x_ref, i_ref, W_ref, lw_ref, o_ref):
    trow = t_ref[0, 0]
    xrow = x_ref[0, 0]
    band = i_ref[0, 0] >> 11
    sub = lax.broadcasted_iota(jnp.int32, (2 * NH, TOK), 0)
    hv = ((sub % NH) + 1).astype(jnp.float32) * (2.0 * math.pi / TMAX)
    th = trow * hv
    F = jnp.where(sub < NH, jnp.sin(th), jnp.cos(th))
    acc = jnp.zeros((2 * EMB, TOK), jnp.float32)
    for i in range(NB):
        mi = (band == i).astype(jnp.float32)
        acc = acc + jnp.dot(W_ref[i], F * mi,
                            preferred_element_type=jnp.float32)
    bsub = lax.broadcasted_iota(jnp.int32, (8, TOK), 0)
    oh = (band == bsub).astype(jnp.float32)
    lsel = jnp.dot(lw_ref[...], oh, preferred_element_type=jnp.float32)
    outT = xrow * lsel * acc[:EMB, :] + acc[EMB:, :]
    o_ref[0] = outT.T


def kernel(x, t, mask, lin_w, a, b, w, v):
    tt = t.transpose(0, 2, 1).reshape(B, N)
    xx = x.transpose(0, 2, 1).reshape(B, N)
    mm = mask.transpose(0, 2, 1).reshape(B, N)
    pad = ((0, 0), (0, NPAD - N))
    tt = jnp.pad(tt, pad).reshape(B, NROW, 128)
    xx = jnp.pad(xx, pad).reshape(B, NROW, 128)
    mm = jnp.pad(mm, pad).reshape(B, NROW, 128)

    blk = pl.BlockSpec((1, NROW, 128), lambda bb: (bb, 0, 0))
    f32 = jnp.float32
    ts, ms, xs, idxs = pl.pallas_call(
        _sort_body,
        grid=(B,),
        in_specs=[blk, blk, blk],
        out_specs=[blk, blk, blk, blk],
        out_shape=[
            jax.ShapeDtypeStruct((B, NROW, 128), f32),
            jax.ShapeDtypeStruct((B, NROW, 128), f32),
            jax.ShapeDtypeStruct((B, NROW, 128), f32),
            jax.ShapeDtypeStruct((B, NROW, 128), jnp.int32),
        ],
    )(tt, mm, xx)

    ts2 = ts.reshape(B, NPAD)[:, :N]
    ms2 = ms.reshape(B, NPAD)[:, :N]
    xs2 = xs.reshape(B, NPAD)[:, :N]
    is2 = idxs.reshape(B, NPAD)[:, :N]

    Wg = jnp.concatenate([a, w], axis=1)
    Wb = jnp.concatenate([b, v], axis=1)
    Wcat = jnp.concatenate([Wg, Wb], axis=2)
    WcatT = Wcat.transpose(0, 2, 1)
    lwT = jnp.zeros((EMB, 8), f32).at[:, :NB].set(lin_w[:, :, 0].T)

    tok_blk = pl.BlockSpec((1, 1, 1, TOK), lambda bb, tb: (bb, tb, 0, 0))
    x_out = pl.pallas_call(
        _film_body,
        grid=(B, NT),
        in_specs=[
            tok_blk, tok_blk, tok_blk,
            pl.BlockSpec((NB, 2 * EMB, 2 * NH), lambda bb, tb: (0, 0, 0)),
            pl.BlockSpec((EMB, 8), lambda bb, tb: (0, 0)),
        ],
        out_specs=pl.BlockSpec((1, TOK, EMB), lambda bb, tb: (bb, tb, 0)),
        out_shape=jax.ShapeDtypeStruct((B, N, EMB), f32),
    )(ts2.reshape(B, NT, 1, TOK), xs2.reshape(B, NT, 1, TOK),
      is2.reshape(B, NT, 1, TOK), WcatT, lwT)

    return (x_out, ms2[..., None], ts2[..., None])
